# baseline (speedup 1.0000x reference)
import functools
import math

import jax
import jax.numpy as jnp
from jax import lax
from jax.experimental import pallas as pl
from jax.experimental.pallas import tpu as pltpu
from jax.experimental.pallas import tpu_sc as plsc

D_MODEL = 64
N_WORKERS = 32
SEQ = 200
BATCHES_PER_WORKER = 128
SPLIT = 128
NBUF = 4
SCALE = math.sqrt(D_MODEL)

_mesh = plsc.VectorSubcoreMesh(core_axis_name="c", subcore_axis_name="s")


@functools.partial(
    pl.kernel,
    mesh=_mesh,
    compiler_params=pltpu.CompilerParams(use_tc_tiling_on_sc=False),
    out_type=jax.ShapeDtypeStruct((N_WORKERS * BATCHES_PER_WORKER, SEQ,
                                   D_MODEL), jnp.float32),
    scratch_types=[
        pltpu.VMEM((BATCHES_PER_WORKER, SEQ), jnp.int32),
        pltpu.VMEM((NBUF, SEQ, D_MODEL), jnp.float32),
        pltpu.SemaphoreType.DMA,
        pltpu.SemaphoreType.DMA,
        pltpu.SemaphoreType.DMA,
        pltpu.SemaphoreType.DMA,
    ],
)
def _embed_sc(x_hbm, table_hbm, out_hbm, idx_v, rows_v, sem0, sem1, sem2,
              sem3):
    wid = lax.axis_index("s") * 2 + lax.axis_index("c")
    base = wid * BATCHES_PER_WORKER
    sems = (sem0, sem1, sem2, sem3)

    pltpu.sync_copy(x_hbm.at[pl.ds(base, BATCHES_PER_WORKER)], idx_v)

    def fire(b, q):
        pltpu.async_copy(
            table_hbm.at[idx_v.at[b, pl.ds(0, SPLIT)]],
            rows_v.at[q, pl.ds(0, SPLIT)], sems[q])
        pltpu.async_copy(
            table_hbm.at[idx_v.at[b, pl.ds(SPLIT, SEQ - SPLIT)]],
            rows_v.at[q, pl.ds(SPLIT, SEQ - SPLIT)], sems[q])

    def drain(q):
        pltpu.make_async_copy(
            table_hbm.at[pl.ds(0, SEQ)], rows_v.at[q], sems[q]).wait()

    for q in range(NBUF):
        fire(q, q)

    def outer(b4, carry):
        for q in range(NBUF):
            b = b4 * NBUF + q
            drain(q)

            buf = rows_v.at[q]

            @plsc.parallel_loop(0, SEQ, step=8, unroll=2)
            def _scale(i):
                for k in range(8):
                    for j in range(D_MODEL // 16):
                        sl = pl.ds(j * 16, 16)
                        buf[i + k, sl] = buf[i + k, sl] * SCALE

            pltpu.sync_copy(buf, out_hbm.at[base + b])

            @pl.when(b + NBUF < BATCHES_PER_WORKER)
            def _():
                fire(b + NBUF, q)
        return carry

    lax.fori_loop(0, BATCHES_PER_WORKER // NBUF, outer, 0)


def kernel(x, table):
    return _embed_sc(x, table)


# speedup vs baseline: 1.0003x; 1.0003x over previous
import functools
import math

import jax
import jax.numpy as jnp
from jax import lax
from jax.experimental import pallas as pl
from jax.experimental.pallas import tpu as pltpu
from jax.experimental.pallas import tpu_sc as plsc

D_MODEL = 64
N_WORKERS = 32
SEQ = 200
BATCHES_PER_WORKER = 128
SPLIT = 128
NBUF = 4
SCALE = math.sqrt(D_MODEL)

_mesh = plsc.VectorSubcoreMesh(core_axis_name="c", subcore_axis_name="s")


@functools.partial(
    pl.kernel,
    mesh=_mesh,
    compiler_params=pltpu.CompilerParams(use_tc_tiling_on_sc=False),
    out_type=jax.ShapeDtypeStruct((N_WORKERS * BATCHES_PER_WORKER, SEQ,
                                   D_MODEL), jnp.float32),
    scratch_types=[
        pltpu.VMEM((BATCHES_PER_WORKER, SEQ), jnp.int32),
        pltpu.VMEM((NBUF, SEQ, D_MODEL), jnp.float32),
        pltpu.SemaphoreType.DMA,
        pltpu.SemaphoreType.DMA,
        pltpu.SemaphoreType.DMA,
        pltpu.SemaphoreType.DMA,
        pltpu.SemaphoreType.DMA,
    ],
)
def _embed_sc(x_hbm, table_hbm, out_hbm, idx_v, rows_v, sem0, sem1, sem2,
              sem3, store_sem):
    wid = lax.axis_index("s") * 2 + lax.axis_index("c")
    base = wid * BATCHES_PER_WORKER
    sems = (sem0, sem1, sem2, sem3)

    pltpu.sync_copy(x_hbm.at[pl.ds(base, BATCHES_PER_WORKER)], idx_v)

    def fire(b, q):
        pltpu.async_copy(
            table_hbm.at[idx_v.at[b, pl.ds(0, SPLIT)]],
            rows_v.at[q, pl.ds(0, SPLIT)], sems[q])
        pltpu.async_copy(
            table_hbm.at[idx_v.at[b, pl.ds(SPLIT, SEQ - SPLIT)]],
            rows_v.at[q, pl.ds(SPLIT, SEQ - SPLIT)], sems[q])

    def drain(q):
        pltpu.make_async_copy(
            table_hbm.at[pl.ds(0, SEQ)], rows_v.at[q], sems[q]).wait()

    def wait_one_store():
        pltpu.make_async_copy(
            rows_v.at[0], out_hbm.at[0], store_sem).wait()

    for q in range(NBUF):
        fire(q, q)

    def outer(b4, carry):
        for q in range(NBUF):
            b = b4 * NBUF + q

            refill = (q - 1) % NBUF
            bq = b4 * NBUF + q - 1

            @pl.when((b >= 1) & (b + NBUF - 1 < BATCHES_PER_WORKER))
            def _():
                wait_one_store()
                fire(bq + NBUF, refill)

            drain(q)

            buf = rows_v.at[q]

            @plsc.parallel_loop(0, SEQ, step=8, unroll=2)
            def _scale(i):
                for k in range(8):
                    for j in range(D_MODEL // 16):
                        sl = pl.ds(j * 16, 16)
                        buf[i + k, sl] = buf[i + k, sl] * SCALE

            pltpu.async_copy(buf, out_hbm.at[base + b], store_sem)
        return carry

    lax.fori_loop(0, BATCHES_PER_WORKER // NBUF, outer, 0)

    for _ in range(NBUF):
        wait_one_store()


def kernel(x, table):
    return _embed_sc(x, table)


# speedup vs baseline: 1.0010x; 1.0007x over previous
import functools
import math

import jax
import jax.numpy as jnp
from jax import lax
from jax.experimental import pallas as pl
from jax.experimental.pallas import tpu as pltpu
from jax.experimental.pallas import tpu_sc as plsc

D_MODEL = 64
N_WORKERS = 32
SEQ = 200
BATCHES_PER_WORKER = 128
SPLIT = 128
NBUF = 4
SCALE = math.sqrt(D_MODEL)

_mesh = plsc.VectorSubcoreMesh(core_axis_name="c", subcore_axis_name="s")


@functools.partial(
    pl.kernel,
    mesh=_mesh,
    compiler_params=pltpu.CompilerParams(use_tc_tiling_on_sc=False),
    out_type=jax.ShapeDtypeStruct((N_WORKERS * BATCHES_PER_WORKER, SEQ,
                                   D_MODEL), jnp.float32),
    scratch_types=[
        pltpu.VMEM((BATCHES_PER_WORKER, SEQ), jnp.int32),
        pltpu.VMEM((NBUF, SEQ, D_MODEL), jnp.float32),
        pltpu.SemaphoreType.DMA,
        pltpu.SemaphoreType.DMA,
        pltpu.SemaphoreType.DMA,
        pltpu.SemaphoreType.DMA,
        pltpu.SemaphoreType.DMA,
    ],
)
def _embed_sc(x_hbm, table_hbm, out_hbm, idx_v, rows_v, sem0, sem1, sem2,
              sem3, store_sem):
    wid = lax.axis_index("s") * 2 + lax.axis_index("c")
    base = wid * BATCHES_PER_WORKER
    sems = (sem0, sem1, sem2, sem3)

    pltpu.sync_copy(x_hbm.at[pl.ds(base, BATCHES_PER_WORKER)], idx_v)

    def fire(b, q):
        pltpu.async_copy(
            table_hbm.at[idx_v.at[b]], rows_v.at[q], sems[q])

    def drain(q):
        pltpu.make_async_copy(
            table_hbm.at[pl.ds(0, SEQ)], rows_v.at[q], sems[q]).wait()

    def wait_one_store():
        pltpu.make_async_copy(
            rows_v.at[0], out_hbm.at[0], store_sem).wait()

    for q in range(NBUF):
        fire(q, q)

    def outer(b4, carry):
        for q in range(NBUF):
            b = b4 * NBUF + q

            refill = (q - 1) % NBUF
            bq = b4 * NBUF + q - 1

            @pl.when((b >= 1) & (b + NBUF - 1 < BATCHES_PER_WORKER))
            def _():
                wait_one_store()
                fire(bq + NBUF, refill)

            drain(q)

            buf = rows_v.at[q]

            @plsc.parallel_loop(0, SEQ, step=8, unroll=2)
            def _scale(i):
                for k in range(8):
                    for j in range(D_MODEL // 16):
                        sl = pl.ds(j * 16, 16)
                        buf[i + k, sl] = buf[i + k, sl] * SCALE

            pltpu.async_copy(buf, out_hbm.at[base + b], store_sem)
        return carry

    lax.fori_loop(0, BATCHES_PER_WORKER // NBUF, outer, 0)

    for _ in range(NBUF):
        wait_one_store()


def kernel(x, table):
    return _embed_sc(x, table)
